# baseline (speedup 1.0000x reference)
import functools

import jax
import jax.numpy as jnp
from jax import lax
from jax.experimental import pallas as pl
from jax.experimental.pallas import tpu as pltpu
from jax.experimental.pallas import tpu_sc as plsc

N = 10000
D = 128
E = 320000
L = 3

NC = 2
NS = 16
NW = NC * NS
EPT = E // NW
CHUNK = 128
EPT_PAD = 10368
NCH = EPT_PAD // CHUNK
NBUF = 3
NPAD = 10112
RPT = NPAD // NS
ZB = RPT // CHUNK
ZTAIL = RPT - ZB * CHUNK

_mesh = plsc.VectorSubcoreMesh(core_axis_name="c", subcore_axis_name="s")


@functools.partial(
    pl.kernel,
    out_type=jax.ShapeDtypeStruct((NC, NPAD, D), jnp.float32),
    mesh=_mesh,
    scratch_types=[
        pltpu.VMEM_SHARED((NPAD, D), jnp.float32),
        pltpu.VMEM((CHUNK, D), jnp.float32),
        pltpu.VMEM((CHUNK, D), jnp.float32),
        pltpu.VMEM((CHUNK, D), jnp.float32),
        pltpu.VMEM((CHUNK,), jnp.int32),
        pltpu.VMEM((CHUNK,), jnp.int32),
        pltpu.VMEM((CHUNK,), jnp.int32),
        pltpu.VMEM((1, CHUNK), jnp.int32),
        pltpu.VMEM((1, CHUNK), jnp.int32),
        pltpu.VMEM((1, CHUNK), jnp.int32),
        pltpu.SemaphoreType.DMA,
        pltpu.SemaphoreType.DMA,
        pltpu.SemaphoreType.DMA,
        pltpu.SemaphoreType.DMA,
    ],
)
def _agg(x_hbm, src_hbm, dst_hbm, out_hbm,
         accum, rows0, rows1, rows2, srcv0, srcv1, srcv2,
         dstv0, dstv1, dstv2, sem0, sem1, sem2, semz):
    c = lax.axis_index("c")
    s = lax.axis_index("s")
    wid = c * NS + s

    @pl.loop(0, CHUNK)
    def _zfill(r):
        @pl.loop(0, D // 16)
        def _zlane(k):
            rows2[r, pl.ds(k * 16, 16)] = jnp.zeros((16,), jnp.float32)

    @pl.loop(0, ZB)
    def _zissue(t):
        pltpu.async_copy(rows2, accum.at[pl.ds(s * RPT + t * CHUNK, CHUNK)], semz)

    pltpu.async_copy(rows2.at[pl.ds(0, ZTAIL)],
                     accum.at[pl.ds(s * RPT + ZB * CHUNK, ZTAIL)], semz)

    base = wid * EPT_PAD
    pltpu.sync_copy(src_hbm.at[pl.ds(base, CHUNK)], srcv0)
    pltpu.sync_copy(dst_hbm.at[pl.ds(base, CHUNK)], dstv0.at[0])
    pltpu.sync_copy(src_hbm.at[pl.ds(base + CHUNK, CHUNK)], srcv1)
    pltpu.sync_copy(dst_hbm.at[pl.ds(base + CHUNK, CHUNK)], dstv1.at[0])
    pltpu.sync_copy(src_hbm.at[pl.ds(base + 2 * CHUNK, CHUNK)], srcv2)
    pltpu.sync_copy(dst_hbm.at[pl.ds(base + 2 * CHUNK, CHUNK)], dstv2.at[0])
    pltpu.async_copy(x_hbm.at[srcv0], rows0, sem0)
    pltpu.async_copy(x_hbm.at[srcv1], rows1, sem1)

    @pl.loop(0, ZB)
    def _zdrain(t):
        pltpu.make_async_copy(rows2, accum.at[pl.ds(s * RPT, CHUNK)], semz).wait()

    pltpu.make_async_copy(rows2.at[pl.ds(0, ZTAIL)],
                          accum.at[pl.ds(s * RPT, ZTAIL)], semz).wait()

    plsc.subcore_barrier()

    pltpu.async_copy(x_hbm.at[srcv2], rows2, sem2)

    @pl.loop(0, NCH, step=NBUF)
    def _edges(j):
        for b, (rows, srcv, dstv, sem) in enumerate((
                (rows0, srcv0, dstv0, sem0), (rows1, srcv1, dstv1, sem1),
                (rows2, srcv2, dstv2, sem2))):
            pltpu.make_async_copy(x_hbm.at[srcv], rows, sem).wait()
            pltpu.sync_copy(rows, accum.at[dstv.at[0]], add=True)

            @pl.when(j + b + NBUF < NCH)
            def _next():
                off = base + (j + b + NBUF) * CHUNK
                pltpu.sync_copy(src_hbm.at[pl.ds(off, CHUNK)], srcv)
                pltpu.sync_copy(dst_hbm.at[pl.ds(off, CHUNK)], dstv.at[0])
                pltpu.async_copy(x_hbm.at[srcv], rows, sem)

    plsc.subcore_barrier()

    pltpu.sync_copy(accum.at[pl.ds(s * RPT, RPT)],
                    out_hbm.at[c, pl.ds(s * RPT, RPT)])


_TC_BLK = 2000


def _gin_tc_body(x_ref, p_ref, w_ref, b_ref, o_ref):
    h = x_ref[...] + p_ref[0] + p_ref[1]
    y = jnp.dot(h, w_ref[...], preferred_element_type=jnp.float32) + b_ref[...]
    o_ref[...] = jnp.maximum(y, 0.0)


def _gin_tc(x, p, w, b):
    return pl.pallas_call(
        _gin_tc_body,
        grid=(N // _TC_BLK,),
        in_specs=[
            pl.BlockSpec((_TC_BLK, D), lambda i: (i, 0)),
            pl.BlockSpec((NC, _TC_BLK, D), lambda i: (0, i, 0)),
            pl.BlockSpec((D, D), lambda i: (0, 0)),
            pl.BlockSpec((1, D), lambda i: (0, 0)),
        ],
        out_specs=pl.BlockSpec((_TC_BLK, D), lambda i: (i, 0)),
        out_shape=jax.ShapeDtypeStruct((N, D), jnp.float32),
    )(x, p, w, b)


def kernel(x, edge_indices, W0, b0, W1, b1, W2, b2):
    Ws = (W0, W1, W2)
    bs = (b0, b1, b2)
    padlen = EPT_PAD - EPT
    iota = jnp.arange(padlen, dtype=jnp.int32)
    pad_src = jnp.broadcast_to(iota * 37 % N, (L, NW, padlen))
    pad_dst = jnp.broadcast_to(N + iota % (NPAD - N), (L, NW, padlen))
    srcs = jnp.concatenate(
        [edge_indices[:, 1, :].reshape(L, NW, EPT), pad_src],
        axis=2).reshape(L, NW * EPT_PAD)
    dsts = jnp.concatenate(
        [edge_indices[:, 0, :].reshape(L, NW, EPT), pad_dst],
        axis=2).reshape(L, NW * EPT_PAD)
    for i in range(L):
        p = _agg(x, srcs[i], dsts[i])
        x = _gin_tc(x, p, Ws[i], bs[i].reshape(1, D))
    return x


# speedup vs baseline: 1.1684x; 1.1684x over previous
import functools

import jax
import jax.numpy as jnp
from jax import lax
from jax.experimental import pallas as pl
from jax.experimental.pallas import tpu as pltpu
from jax.experimental.pallas import tpu_sc as plsc

N = 10000
D = 128
E = 320000
L = 3

NC = 2
NS = 16
NW = NC * NS
EPT = E // NW
CHUNK = 96
EPT_PAD = 10368
NCH = EPT_PAD // CHUNK
NBUF = 4
NPAD = 10112
RPT = NPAD // NS
ZB = RPT // CHUNK
ZTAIL = RPT - ZB * CHUNK

_mesh = plsc.VectorSubcoreMesh(core_axis_name="c", subcore_axis_name="s")


@functools.partial(
    pl.kernel,
    out_type=jax.ShapeDtypeStruct((NC, NPAD, D), jnp.float32),
    mesh=_mesh,
    scratch_types=[
        pltpu.VMEM_SHARED((NPAD, D), jnp.float32),
        pltpu.VMEM((CHUNK, D), jnp.float32),
        pltpu.VMEM((CHUNK, D), jnp.float32),
        pltpu.VMEM((CHUNK, D), jnp.float32),
        pltpu.VMEM((CHUNK, D), jnp.float32),
        pltpu.VMEM((CHUNK,), jnp.int32),
        pltpu.VMEM((CHUNK,), jnp.int32),
        pltpu.VMEM((CHUNK,), jnp.int32),
        pltpu.VMEM((CHUNK,), jnp.int32),
        pltpu.VMEM((1, CHUNK), jnp.int32),
        pltpu.VMEM((1, CHUNK), jnp.int32),
        pltpu.VMEM((1, CHUNK), jnp.int32),
        pltpu.VMEM((1, CHUNK), jnp.int32),
        pltpu.SemaphoreType.DMA,
        pltpu.SemaphoreType.DMA,
        pltpu.SemaphoreType.DMA,
        pltpu.SemaphoreType.DMA,
        pltpu.SemaphoreType.DMA,
        pltpu.SemaphoreType.DMA,
        pltpu.SemaphoreType.DMA,
        pltpu.SemaphoreType.DMA,
        pltpu.SemaphoreType.DMA,
    ],
)
def _agg(x_hbm, src_hbm, dst_hbm, out_hbm, accum,
         rows0, rows1, rows2, rows3, srcv0, srcv1, srcv2, srcv3,
         dstv0, dstv1, dstv2, dstv3, g0, g1, g2, g3, s0, s1, s2, s3, semz):
    c = lax.axis_index("c")
    s = lax.axis_index("s")
    wid = c * NS + s

    ROWS = (rows0, rows1, rows2, rows3)
    SRCV = (srcv0, srcv1, srcv2, srcv3)
    DSTV = (dstv0, dstv1, dstv2, dstv3)
    GSEM = (g0, g1, g2, g3)
    SSEM = (s0, s1, s2, s3)

    @pl.loop(0, CHUNK)
    def _zfill(r):
        @pl.loop(0, D // 16)
        def _zlane(k):
            rows3[r, pl.ds(k * 16, 16)] = jnp.zeros((16,), jnp.float32)

    @pl.loop(0, ZB)
    def _zissue(t):
        pltpu.async_copy(rows3, accum.at[pl.ds(s * RPT + t * CHUNK, CHUNK)], semz)

    pltpu.async_copy(rows3.at[pl.ds(0, ZTAIL)],
                     accum.at[pl.ds(s * RPT + ZB * CHUNK, ZTAIL)], semz)

    base = wid * EPT_PAD
    pltpu.sync_copy(src_hbm.at[pl.ds(base, CHUNK)], srcv0)
    pltpu.sync_copy(dst_hbm.at[pl.ds(base, CHUNK)], dstv0.at[0])
    pltpu.sync_copy(src_hbm.at[pl.ds(base + CHUNK, CHUNK)], srcv1)
    pltpu.sync_copy(dst_hbm.at[pl.ds(base + CHUNK, CHUNK)], dstv1.at[0])
    pltpu.async_copy(x_hbm.at[srcv0], rows0, g0)
    pltpu.async_copy(x_hbm.at[srcv1], rows1, g1)

    @pl.loop(0, ZB)
    def _zdrain(t):
        pltpu.make_async_copy(rows3, accum.at[pl.ds(s * RPT, CHUNK)], semz).wait()

    pltpu.make_async_copy(rows3.at[pl.ds(0, ZTAIL)],
                          accum.at[pl.ds(s * RPT, ZTAIL)], semz).wait()

    plsc.subcore_barrier()

    @pl.loop(0, NCH, step=NBUF)
    def _edges(j):
        for bb in range(NBUF):
            cc = j + bb
            rows, srcv, dstv = ROWS[bb], SRCV[bb], DSTV[bb]
            nb = (bb + 2) % NBUF
            nrows, nsrcv, ndstv = ROWS[nb], SRCV[nb], DSTV[nb]

            @pl.when(cc >= 2)
            def _ws():
                pltpu.make_async_copy(
                    nrows, accum.at[ndstv.at[0]], SSEM[nb]).wait()

            @pl.when(cc + 2 < NCH)
            def _ig():
                off = base + (cc + 2) * CHUNK
                pltpu.sync_copy(src_hbm.at[pl.ds(off, CHUNK)], nsrcv)
                pltpu.sync_copy(dst_hbm.at[pl.ds(off, CHUNK)], ndstv.at[0])
                pltpu.async_copy(x_hbm.at[nsrcv], nrows, GSEM[nb])

            pltpu.make_async_copy(x_hbm.at[srcv], rows, GSEM[bb]).wait()
            pltpu.async_copy(rows, accum.at[dstv.at[0]], SSEM[bb], add=True)

    for bb in ((NCH - 2) % NBUF, (NCH - 1) % NBUF):
        pltpu.make_async_copy(ROWS[bb], accum.at[DSTV[bb].at[0]],
                              SSEM[bb]).wait()

    plsc.subcore_barrier()

    pltpu.sync_copy(accum.at[pl.ds(s * RPT, RPT)],
                    out_hbm.at[c, pl.ds(s * RPT, RPT)])


_TC_BLK = 2000


def _gin_tc_body(x_ref, p_ref, w_ref, b_ref, o_ref):
    h = x_ref[...] + p_ref[0] + p_ref[1]
    y = jnp.dot(h, w_ref[...], preferred_element_type=jnp.float32) + b_ref[...]
    o_ref[...] = jnp.maximum(y, 0.0)


def _gin_tc(x, p, w, b):
    return pl.pallas_call(
        _gin_tc_body,
        grid=(N // _TC_BLK,),
        in_specs=[
            pl.BlockSpec((_TC_BLK, D), lambda i: (i, 0)),
            pl.BlockSpec((NC, _TC_BLK, D), lambda i: (0, i, 0)),
            pl.BlockSpec((D, D), lambda i: (0, 0)),
            pl.BlockSpec((1, D), lambda i: (0, 0)),
        ],
        out_specs=pl.BlockSpec((_TC_BLK, D), lambda i: (i, 0)),
        out_shape=jax.ShapeDtypeStruct((N, D), jnp.float32),
    )(x, p, w, b)


def kernel(x, edge_indices, W0, b0, W1, b1, W2, b2):
    Ws = (W0, W1, W2)
    bs = (b0, b1, b2)
    padlen = EPT_PAD - EPT
    iota = jnp.arange(padlen, dtype=jnp.int32)
    pad_src = jnp.broadcast_to(iota * 37 % N, (L, NW, padlen))
    pad_dst = jnp.broadcast_to(N + iota % (NPAD - N), (L, NW, padlen))
    srcs = jnp.concatenate(
        [edge_indices[:, 1, :].reshape(L, NW, EPT), pad_src],
        axis=2).reshape(L, NW * EPT_PAD)
    dsts = jnp.concatenate(
        [edge_indices[:, 0, :].reshape(L, NW, EPT), pad_dst],
        axis=2).reshape(L, NW * EPT_PAD)
    for i in range(L):
        p = _agg(x, srcs[i], dsts[i])
        x = _gin_tc(x, p, Ws[i], bs[i].reshape(1, D))
    return x


# speedup vs baseline: 1.3707x; 1.1732x over previous
import functools

import jax
import jax.numpy as jnp
from jax import lax
from jax.experimental import pallas as pl
from jax.experimental.pallas import tpu as pltpu
from jax.experimental.pallas import tpu_sc as plsc

N = 10000
D = 128
E = 320000
L = 3

NC = 2
NS = 16
NW = NC * NS
EPT = E // NW
CHUNK = 96
EPT_PAD = 10368
NCH = EPT_PAD // CHUNK
NBUF = 4
NPAD = 10112
RPT = NPAD // NS
ZB = RPT // CHUNK
ZTAIL = RPT - ZB * CHUNK

_mesh = plsc.VectorSubcoreMesh(core_axis_name="c", subcore_axis_name="s")


@functools.partial(
    pl.kernel,
    out_type=jax.ShapeDtypeStruct((NC, NPAD, D), jnp.float32),
    mesh=_mesh,
    scratch_types=[
        pltpu.VMEM_SHARED((NPAD, D), jnp.float32),
        pltpu.VMEM((CHUNK, D), jnp.float32),
        pltpu.VMEM((CHUNK, D), jnp.float32),
        pltpu.VMEM((CHUNK, D), jnp.float32),
        pltpu.VMEM((CHUNK, D), jnp.float32),
        pltpu.VMEM((CHUNK,), jnp.int32),
        pltpu.VMEM((CHUNK,), jnp.int32),
        pltpu.VMEM((CHUNK,), jnp.int32),
        pltpu.VMEM((CHUNK,), jnp.int32),
        pltpu.VMEM((1, CHUNK), jnp.int32),
        pltpu.VMEM((1, CHUNK), jnp.int32),
        pltpu.VMEM((1, CHUNK), jnp.int32),
        pltpu.VMEM((1, CHUNK), jnp.int32),
        pltpu.SemaphoreType.DMA,
        pltpu.SemaphoreType.DMA,
        pltpu.SemaphoreType.DMA,
        pltpu.SemaphoreType.DMA,
        pltpu.SemaphoreType.DMA,
        pltpu.SemaphoreType.DMA,
        pltpu.SemaphoreType.DMA,
        pltpu.SemaphoreType.DMA,
        pltpu.SemaphoreType.DMA,
        pltpu.SemaphoreType.DMA,
    ],
)
def _agg(x_hbm, src_hbm, dst_hbm, out_hbm, accum,
         rows0, rows1, rows2, rows3, srcv0, srcv1, srcv2, srcv3,
         dstv0, dstv1, dstv2, dstv3, g0, g1, g2, g3, s0, s1, s2, s3, semz,
         semi):
    c = lax.axis_index("c")
    s = lax.axis_index("s")
    wid = c * NS + s

    ROWS = (rows0, rows1, rows2, rows3)
    SRCV = (srcv0, srcv1, srcv2, srcv3)
    DSTV = (dstv0, dstv1, dstv2, dstv3)
    GSEM = (g0, g1, g2, g3)
    SSEM = (s0, s1, s2, s3)

    @pl.loop(0, CHUNK)
    def _zfill(r):
        @pl.loop(0, D // 16)
        def _zlane(k):
            rows3[r, pl.ds(k * 16, 16)] = jnp.zeros((16,), jnp.float32)

    @pl.loop(0, ZB)
    def _zissue(t):
        pltpu.async_copy(rows3, accum.at[pl.ds(s * RPT + t * CHUNK, CHUNK)], semz)

    pltpu.async_copy(rows3.at[pl.ds(0, ZTAIL)],
                     accum.at[pl.ds(s * RPT + ZB * CHUNK, ZTAIL)], semz)

    base = wid * EPT_PAD
    pltpu.sync_copy(src_hbm.at[pl.ds(base, CHUNK)], srcv0)
    pltpu.sync_copy(dst_hbm.at[pl.ds(base, CHUNK)], dstv0.at[0])
    pltpu.sync_copy(src_hbm.at[pl.ds(base + CHUNK, CHUNK)], srcv1)
    pltpu.sync_copy(dst_hbm.at[pl.ds(base + CHUNK, CHUNK)], dstv1.at[0])
    pltpu.async_copy(x_hbm.at[srcv0], rows0, g0)
    pltpu.async_copy(x_hbm.at[srcv1], rows1, g1)

    @pl.loop(0, ZB)
    def _zdrain(t):
        pltpu.make_async_copy(rows3, accum.at[pl.ds(s * RPT, CHUNK)], semz).wait()

    pltpu.make_async_copy(rows3.at[pl.ds(0, ZTAIL)],
                          accum.at[pl.ds(s * RPT, ZTAIL)], semz).wait()

    plsc.subcore_barrier()

    @pl.loop(0, NCH, step=NBUF)
    def _edges(j):
        for bb in range(NBUF):
            cc = j + bb
            rows, srcv, dstv = ROWS[bb], SRCV[bb], DSTV[bb]
            nb = (bb + 2) % NBUF
            nrows, nsrcv, ndstv = ROWS[nb], SRCV[nb], DSTV[nb]

            @pl.when(cc >= 2)
            def _ws():
                pltpu.make_async_copy(
                    nrows, accum.at[ndstv.at[0]], SSEM[nb]).wait()

            @pl.when(cc + 2 < NCH)
            def _ig():
                off = base + (cc + 2) * CHUNK
                ca = pltpu.async_copy(src_hbm.at[pl.ds(off, CHUNK)], nsrcv, semi)
                cb = pltpu.async_copy(dst_hbm.at[pl.ds(off, CHUNK)],
                                      ndstv.at[0], semi)
                ca.wait()
                cb.wait()
                pltpu.async_copy(x_hbm.at[nsrcv], nrows, GSEM[nb])

            pltpu.make_async_copy(x_hbm.at[srcv], rows, GSEM[bb]).wait()
            pltpu.async_copy(rows, accum.at[dstv.at[0]], SSEM[bb], add=True)

    for bb in ((NCH - 2) % NBUF, (NCH - 1) % NBUF):
        pltpu.make_async_copy(ROWS[bb], accum.at[DSTV[bb].at[0]],
                              SSEM[bb]).wait()

    plsc.subcore_barrier()

    pltpu.sync_copy(accum.at[pl.ds(s * RPT, RPT)],
                    out_hbm.at[c, pl.ds(s * RPT, RPT)])


_TC_BLK = 2000


def _gin_tc_body(x_ref, p_ref, w_ref, b_ref, o_ref):
    h = x_ref[...] + p_ref[0] + p_ref[1]
    y = jnp.dot(h, w_ref[...], preferred_element_type=jnp.float32) + b_ref[...]
    o_ref[...] = jnp.maximum(y, 0.0)


def _gin_tc(x, p, w, b):
    return pl.pallas_call(
        _gin_tc_body,
        grid=(N // _TC_BLK,),
        in_specs=[
            pl.BlockSpec((_TC_BLK, D), lambda i: (i, 0)),
            pl.BlockSpec((NC, _TC_BLK, D), lambda i: (0, i, 0)),
            pl.BlockSpec((D, D), lambda i: (0, 0)),
            pl.BlockSpec((1, D), lambda i: (0, 0)),
        ],
        out_specs=pl.BlockSpec((_TC_BLK, D), lambda i: (i, 0)),
        out_shape=jax.ShapeDtypeStruct((N, D), jnp.float32),
    )(x, p, w, b)


def kernel(x, edge_indices, W0, b0, W1, b1, W2, b2):
    Ws = (W0, W1, W2)
    bs = (b0, b1, b2)
    padlen = EPT_PAD - EPT
    iota = jnp.arange(padlen, dtype=jnp.int32)
    pad_src = jnp.broadcast_to(iota * 37 % N, (L, NW, padlen))
    pad_dst = jnp.broadcast_to(N + iota % (NPAD - N), (L, NW, padlen))
    srcs = jnp.concatenate(
        [edge_indices[:, 1, :].reshape(L, NW, EPT), pad_src],
        axis=2).reshape(L, NW * EPT_PAD)
    dsts = jnp.concatenate(
        [edge_indices[:, 0, :].reshape(L, NW, EPT), pad_dst],
        axis=2).reshape(L, NW * EPT_PAD)
    for i in range(L):
        p = _agg(x, srcs[i], dsts[i])
        x = _gin_tc(x, p, Ws[i], bs[i].reshape(1, D))
    return x
